# baseline (speedup 1.0000x reference)
import jax
import jax.numpy as jnp
from jax import lax
from jax.experimental import pallas as pl
from jax.experimental.pallas import tpu as pltpu
from jax.experimental.pallas import tpu_sc as plsc

N_TABLE = 10000
N_COL = 50000
ALPHA = 0.2
K_APPNP = 10

NC, NS = 2, 16
CH = 64
CHW = 64
ZB = 32

NACC_T = 10752
WINROWS = 12544
NACC_W = 12800
BLK = 1792

_MESH = plsc.VectorSubcoreMesh(core_axis_name="c", subcore_axis_name="s")


def _pad_edges(src, dst, dump):
    e = src.shape[0]
    epad = -(-e // 4096) * 4096
    if epad != e:
        pad = epad - e
        src = jnp.concatenate([src, jnp.zeros((pad,), jnp.int32)])
        dst = jnp.concatenate([dst, jnp.full((pad,), dump, jnp.int32)])
    return src, dst


def _fill_rows(ref, nrows, ncols, value):
    v16 = jnp.full((16,), value, jnp.float32)

    def row(r, carry):
        for k in range(ncols // 16):
            ref[r, pl.ds(16 * k, 16)] = v16
        return carry

    lax.fori_loop(0, nrows, row, 0)


def _zero_stripe(zbuf, ref, base, nrows):
    off = 0
    while off < nrows:
        sz = min(ZB, nrows - off)
        pltpu.sync_copy(zbuf.at[pl.ds(0, sz)], ref.at[pl.ds(base + off, sz)])
        off += sz


def _seg_table(feat, src, dst):
    epad = src.shape[0]
    epw = epad // (NC * NS)
    nch = epw // CH
    rpt = NACC_T // NS

    out_type = jax.ShapeDtypeStruct((NC, NACC_T, 128), jnp.float32)
    scratch = [
        pltpu.VMEM_SHARED((NACC_T, 128), jnp.float32),
        pltpu.VMEM((CH,), jnp.int32),
        pltpu.VMEM((CH,), jnp.int32),
        pltpu.VMEM((CH, 128), jnp.float32),
        pltpu.VMEM((ZB, 128), jnp.float32),
        pltpu.SemaphoreType.DMA,
    ]

    def body(feat_h, src_h, dst_h, out_h, accum, srcv, dstv, rows, zbuf, sem):
        c = lax.axis_index("c")
        s = lax.axis_index("s")
        _fill_rows(zbuf, ZB, 128, 0.0)
        base = s * rpt
        _zero_stripe(zbuf, accum, base, rpt)
        plsc.subcore_barrier()

        ebase = c * (epad // 2) + s * epw

        def chunk(i, carry):
            off = ebase + i * CH
            pltpu.sync_copy(src_h.at[pl.ds(off, CH)], srcv)
            pltpu.sync_copy(dst_h.at[pl.ds(off, CH)], dstv)
            pltpu.async_copy(feat_h.at[srcv], rows, sem).wait()
            pltpu.sync_copy(rows, accum.at[dstv], add=True)
            return carry

        lax.fori_loop(0, nch, chunk, 0)
        plsc.subcore_barrier()
        pltpu.sync_copy(accum.at[pl.ds(base, rpt)], out_h.at[c, pl.ds(base, rpt)])

    fn = pl.kernel(body, out_type=out_type, mesh=_MESH, scratch_types=scratch)
    return fn(feat, src, dst)


def _seg_win(feat, src, dst):
    epad = src.shape[0]
    epw = epad // NS
    nch = epw // CHW
    rpt = NACC_W // NS

    out_type = jax.ShapeDtypeStruct((4, NACC_W, 128), jnp.float32)
    scratch = [
        pltpu.VMEM_SHARED((NACC_W, 128), jnp.float32),
        pltpu.VMEM((CHW,), jnp.int32),
        pltpu.VMEM((CHW,), jnp.int32),
        pltpu.VMEM((CHW,), jnp.int32),
        pltpu.VMEM((CHW, 128), jnp.float32),
        pltpu.VMEM((ZB, 128), jnp.float32),
        pltpu.SemaphoreType.DMA,
    ]

    def body(feat_h, src_h, dst_h, out_h,
             accum, srcv, dstv, liv, rows, zbuf, sem):
        c = lax.axis_index("c")
        s = lax.axis_index("s")
        _fill_rows(zbuf, ZB, 128, 0.0)
        base = s * rpt

        for w in range(2):
            wabs = c * 2 + w
            wbase = wabs * WINROWS
            _zero_stripe(zbuf, accum, base, rpt)
            plsc.subcore_barrier()

            def chunk(i, carry):
                off = s * epw + i * CHW
                pltpu.sync_copy(src_h.at[pl.ds(off, CHW)], srcv)
                pltpu.sync_copy(dst_h.at[pl.ds(off, CHW)], dstv)
                for k in range(CHW // 16):
                    d16 = dstv[pl.ds(16 * k, 16)]
                    li = d16 - wbase
                    inw = (li >= 0) & (li < WINROWS)
                    liv[pl.ds(16 * k, 16)] = jnp.where(inw, li, WINROWS)
                pltpu.async_copy(feat_h.at[srcv], rows, sem).wait()
                pltpu.sync_copy(rows, accum.at[liv], add=True)
                return carry

            lax.fori_loop(0, nch, chunk, 0)
            plsc.subcore_barrier()
            pltpu.sync_copy(accum.at[pl.ds(base, rpt)],
                            out_h.at[wabs, pl.ds(base, rpt)])
            plsc.subcore_barrier()

    fn = pl.kernel(body, out_type=out_type, mesh=_MESH, scratch_types=scratch)
    return fn(feat, src, dst)


def _tc_table_proj(x_table, wcat, qv2, wq, bq2):
    grid = (6,)

    def body(x_ref, w_ref, qv_ref, wq_ref, bq_ref, o1, o2, o3, o4):
        x = x_ref[...]
        acc = jnp.dot(x, w_ref[...], preferred_element_type=jnp.float32)
        o1[...] = acc[:, 0:128]
        o2[...] = acc[:, 128:256]
        o3[...] = acc[:, 256:384]
        qv = qv_ref[...]
        q = jnp.dot(qv, wq_ref[...], preferred_element_type=jnp.float32) + bq_ref[...]
        rown = jnp.sqrt(jnp.sum(x * x, axis=1, keepdims=True))
        nq = jnp.sqrt(jnp.sum(qv * qv))
        wts = jnp.maximum(
            acc[:, 384:385] / (jnp.maximum(rown, 1e-12) * jnp.maximum(nq, 1e-12)),
            0.0)
        o4[...] = wts * q

    ospec = pl.BlockSpec((BLK, 128), lambda i: (i, 0))
    return pl.pallas_call(
        body,
        grid=grid,
        in_specs=[
            pl.BlockSpec((BLK, 512), lambda i: (i, 0)),
            pl.BlockSpec((512, 512), lambda i: (0, 0)),
            pl.BlockSpec((1, 512), lambda i: (0, 0)),
            pl.BlockSpec((512, 128), lambda i: (0, 0)),
            pl.BlockSpec((1, 128), lambda i: (0, 0)),
        ],
        out_specs=[ospec] * 4,
        out_shape=[jax.ShapeDtypeStruct((N_TABLE, 128), jnp.float32)] * 4,
    )(x_table, wcat, qv2, wq, bq2)


def _tc_col_proj(x_column, wcat):
    grid = (28,)

    def body(x_ref, w_ref, o1, o2, o3):
        acc = jnp.dot(x_ref[...], w_ref[...], preferred_element_type=jnp.float32)
        o1[...] = acc[:, 0:128]
        o2[...] = acc[:, 128:256]
        o3[...] = acc[:, 256:384]

    ospec = pl.BlockSpec((BLK, 128), lambda i: (i, 0))
    return pl.pallas_call(
        body,
        grid=grid,
        in_specs=[
            pl.BlockSpec((BLK, 128), lambda i: (i, 0)),
            pl.BlockSpec((128, 384), lambda i: (0, 0)),
        ],
        out_specs=[ospec] * 3,
        out_shape=[jax.ShapeDtypeStruct((N_COL, 128), jnp.float32)] * 3,
    )(x_column, wcat)


def _tc_col1(sum_hc, cnt_hc, sum_cs, cnt_cs, xcr, bias, w):
    grid = (28,)

    def body(sh_ref, ch_ref, ss_ref, cs_ref, x_ref, b_ref, w_ref, out):
        chc = ch_ref[0, :, 0:1]
        ccs = cs_ref[0, :, 0:1]
        mhc = sh_ref[0] / jnp.maximum(chc, 1.0)
        mcs = ss_ref[0] / jnp.maximum(ccs, 1.0)
        col1 = jnp.maximum(mhc + mcs + x_ref[...] + b_ref[...], 0.0)
        out[...] = jnp.dot(col1, w_ref[...], preferred_element_type=jnp.float32)

    sspec = pl.BlockSpec((1, BLK, 128), lambda i: (i // 7, i % 7, 0))
    cspec = sspec
    return pl.pallas_call(
        body,
        grid=grid,
        in_specs=[
            sspec, cspec, sspec, cspec,
            pl.BlockSpec((BLK, 128), lambda i: (i, 0)),
            pl.BlockSpec((1, 128), lambda i: (0, 0)),
            pl.BlockSpec((128, 128), lambda i: (0, 0)),
        ],
        out_specs=pl.BlockSpec((BLK, 128), lambda i: (i, 0)),
        out_shape=jax.ShapeDtypeStruct((N_COL, 128), jnp.float32),
    )(sum_hc, cnt_hc, sum_cs, cnt_cs, xcr, bias, w)


def _tc_tab1(sum_rev, cnt_rev, sum_tt, cnt_tt, xtr, bias, wcat):
    grid = (6,)

    def body(sr_ref, cr_ref, st_ref, ct_ref, x_ref, b_ref, w_ref, o1, o2):
        cr = cr_ref[0, :, 0:1] + cr_ref[1, :, 0:1]
        ct = ct_ref[0, :, 0:1] + ct_ref[1, :, 0:1]
        mr = (sr_ref[0] + sr_ref[1]) / jnp.maximum(cr, 1.0)
        mt = (st_ref[0] + st_ref[1]) / jnp.maximum(ct, 1.0)
        tab1 = jnp.maximum(mr + mt + x_ref[...] + b_ref[...], 0.0)
        acc = jnp.dot(tab1, w_ref[...], preferred_element_type=jnp.float32)
        o1[...] = acc[:, 0:128]
        o2[...] = acc[:, 128:256]

    sspec = pl.BlockSpec((2, BLK, 128), lambda i: (0, i, 0))
    cspec = pl.BlockSpec((2, BLK, 128), lambda i: (0, i, 0))
    ospec = pl.BlockSpec((BLK, 128), lambda i: (i, 0))
    return pl.pallas_call(
        body,
        grid=grid,
        in_specs=[
            sspec, cspec, sspec, cspec,
            pl.BlockSpec((BLK, 128), lambda i: (i, 0)),
            pl.BlockSpec((1, 128), lambda i: (0, 0)),
            pl.BlockSpec((128, 256), lambda i: (0, 0)),
        ],
        out_specs=[ospec] * 2,
        out_shape=[jax.ShapeDtypeStruct((N_TABLE, 128), jnp.float32)] * 2,
    )(sum_rev, cnt_rev, sum_tt, cnt_tt, xtr, bias, wcat)


def _tc_x0(s2rev, cnt_rev, s2tt, cnt_tt, tab1r, z0, bias):
    grid = (6,)

    def body(sr_ref, cr_ref, st_ref, ct_ref, x_ref, z_ref, b_ref, oh, oy):
        cr = cr_ref[0, :, 0:1] + cr_ref[1, :, 0:1]
        ct = ct_ref[0, :, 0:1] + ct_ref[1, :, 0:1]
        mr = (sr_ref[0] + sr_ref[1]) / jnp.maximum(cr, 1.0)
        mt = (st_ref[0] + st_ref[1]) / jnp.maximum(ct, 1.0)
        x0 = mr + mt + x_ref[...] + b_ref[...] + z_ref[...]
        dinv = lax.rsqrt(ct + 1.0)
        oh[...] = x0
        oy[...] = dinv * x0

    sspec = pl.BlockSpec((2, BLK, 128), lambda i: (0, i, 0))
    cspec = pl.BlockSpec((2, BLK, 128), lambda i: (0, i, 0))
    ospec = pl.BlockSpec((BLK, 128), lambda i: (i, 0))
    return pl.pallas_call(
        body,
        grid=grid,
        in_specs=[
            sspec, cspec, sspec, cspec,
            pl.BlockSpec((BLK, 128), lambda i: (i, 0)),
            pl.BlockSpec((BLK, 128), lambda i: (i, 0)),
            pl.BlockSpec((1, 128), lambda i: (0, 0)),
        ],
        out_specs=[ospec] * 2,
        out_shape=[jax.ShapeDtypeStruct((N_TABLE, 128), jnp.float32)] * 2,
    )(s2rev, cnt_rev, s2tt, cnt_tt, tab1r, z0, bias)


def _tc_appnp_step(part, cnt_tt, y, h):
    grid = (6,)

    def body(p_ref, ct_ref, y_ref, h_ref, oz, oy):
        ct = ct_ref[0, :, 0:1] + ct_ref[1, :, 0:1]
        dinv = lax.rsqrt(ct + 1.0)
        psum = p_ref[0] + p_ref[1]
        z = (1.0 - ALPHA) * dinv * (psum + y_ref[...]) + ALPHA * h_ref[...]
        oz[...] = z
        oy[...] = dinv * z

    sspec = pl.BlockSpec((2, BLK, 128), lambda i: (0, i, 0))
    cspec = pl.BlockSpec((2, BLK, 128), lambda i: (0, i, 0))
    ospec = pl.BlockSpec((BLK, 128), lambda i: (i, 0))
    return pl.pallas_call(
        body,
        grid=grid,
        in_specs=[
            sspec, cspec,
            pl.BlockSpec((BLK, 128), lambda i: (i, 0)),
            pl.BlockSpec((BLK, 128), lambda i: (i, 0)),
        ],
        out_specs=[ospec] * 2,
        out_shape=[jax.ShapeDtypeStruct((N_TABLE, 128), jnp.float32)] * 2,
    )(part, cnt_tt, y, h)


def _tc_scores(z, wo_row, bo):
    grid = (6,)

    def body(z_ref, w_ref, b_ref, out):
        s = jnp.sum(z_ref[...] * w_ref[...], axis=1, keepdims=True) + b_ref[...]
        out[...] = jnp.broadcast_to(s, (BLK, 128))

    return pl.pallas_call(
        body,
        grid=grid,
        in_specs=[
            pl.BlockSpec((BLK, 128), lambda i: (i, 0)),
            pl.BlockSpec((1, 128), lambda i: (0, 0)),
            pl.BlockSpec((1, 1), lambda i: (0, 0)),
        ],
        out_specs=pl.BlockSpec((BLK, 128), lambda i: (i, 0)),
        out_shape=jax.ShapeDtypeStruct((N_TABLE, 128), jnp.float32),
    )(z, wo_row, bo)


def kernel(x_table, x_column, query_vec, hc_src, hc_dst, rev_src, rev_dst,
           cs_src, cs_dst, tt_src, tt_dst, params):
    p = params
    f32 = jnp.float32

    wcat1 = jnp.concatenate(
        [p["Wl1_hc"], p["Wl1_tt"], p["Wr1_rev"] + p["Wr1_tt"],
         query_vec[:, None], jnp.zeros((512, 127), f32)], axis=1)
    qv2 = query_vec[None, :]
    bq2 = p["bq"][None, :]
    o_hc, o_tt, xtr, z0 = _tc_table_proj(x_table, wcat1, qv2, p["Wq"], bq2)

    wcat2 = jnp.concatenate(
        [p["Wl1_cs"], p["Wl1_rev"], p["Wr1_hc"] + p["Wr1_cs"]], axis=1)
    o_cs, o_rev, xcr = _tc_col_proj(x_column, wcat2)

    hs, hd = _pad_edges(hc_src, hc_dst, N_COL)
    css, csd = _pad_edges(cs_src, cs_dst, N_COL)
    rs, rd = _pad_edges(rev_src, rev_dst, N_TABLE)
    ts, td = _pad_edges(tt_src, tt_dst, N_TABLE)

    onesf = jnp.ones((8, 128), f32)
    cnt_hc = _seg_win(onesf, jnp.zeros_like(hd), hd)
    cnt_cs = _seg_win(onesf, jnp.zeros_like(csd), csd)
    cnt_rev = _seg_table(onesf, jnp.zeros_like(rd), rd)
    cnt_tt = _seg_table(onesf, jnp.zeros_like(td), td)
    sum_hc = _seg_win(o_hc, hs, hd)
    sum_cs = _seg_win(o_cs, css, csd)
    sum_rev = _seg_table(o_rev, rs, rd)
    sum_tt = _seg_table(o_tt, ts, td)

    bias1c = (p["b1_hc"] + p["b1_cs"])[None, :]
    col1p = _tc_col1(sum_hc, cnt_hc, sum_cs, cnt_cs, xcr, bias1c, p["Wl2_rev"])

    bias1t = (p["b1_rev"] + p["b1_tt"])[None, :]
    wcat3 = jnp.concatenate([p["Wl2_tt"], p["Wr2_rev"] + p["Wr2_tt"]], axis=1)
    tab1p, tab1r = _tc_tab1(sum_rev, cnt_rev, sum_tt, cnt_tt, xtr, bias1t, wcat3)

    s2rev = _seg_table(col1p, rs, rd)
    s2tt = _seg_table(tab1p, ts, td)

    bias2t = (p["b2_rev"] + p["b2_tt"])[None, :]
    h, y = _tc_x0(s2rev, cnt_rev, s2tt, cnt_tt, tab1r, z0, bias2t)

    z = h
    for _ in range(K_APPNP):
        part = _seg_table(y, ts, td)
        z, y = _tc_appnp_step(part, cnt_tt, y, h)

    wo_row = p["Wo"].reshape(1, 128)
    bo = p["bo"].reshape(1, 1)
    sb = _tc_scores(z, wo_row, bo)
    return sb[:, 0]


# speedup vs baseline: 1.0171x; 1.0171x over previous
import jax
import jax.numpy as jnp
from jax import lax
from jax.experimental import pallas as pl
from jax.experimental.pallas import tpu as pltpu
from jax.experimental.pallas import tpu_sc as plsc

N_TABLE = 10000
N_COL = 50000
ALPHA = 0.2
K_APPNP = 10

NC, NS = 2, 16
CHK = 64
ZB = 32

NACC_T = 10752
WINROWS = 12544
NACC_W = 12800
BLK = 1792

_MESH = plsc.VectorSubcoreMesh(core_axis_name="c", subcore_axis_name="s")


def _pad_edges(src, dst, dump, mult):
    e = src.shape[0]
    epad = -(-e // mult) * mult
    if epad != e:
        pad = epad - e
        src = jnp.concatenate([src, jnp.zeros((pad,), jnp.int32)])
        dst = jnp.concatenate([dst, jnp.full((pad,), dump, jnp.int32)])
    return src.reshape(-1, CHK), dst.reshape(-1, CHK)


def _fill_rows(ref, nrows, ncols, value):
    v16 = jnp.full((16,), value, jnp.float32)

    def row(r, carry):
        for k in range(ncols // 16):
            ref[r, pl.ds(16 * k, 16)] = v16
        return carry

    lax.fori_loop(0, nrows, row, 0)


def _zero_stripe(zbuf, ref, base, nrows, semz):
    cps = []
    off = 0
    while off < nrows:
        sz = min(ZB, nrows - off)
        cps.append(pltpu.async_copy(zbuf.at[pl.ds(0, sz)],
                                    ref.at[pl.ds(base + off, sz)], semz))
        off += sz
    for cp in cps:
        cp.wait()


def _pipelined_sweep(feat_h, src2, dst2, accum, s, row_base, nchk, refs,
                     wbase=None):
    (rowsA, rowsB, idxSA, idxSB, idxDA, idxDB,
     semI, semG0, semG1, semS0, semS1) = refs
    nit = nchk // 4
    total_rows = src2.shape[0]

    def make_liv(idxD, b):
        if wbase is None:
            return idxD.at[b]
        for k in range(CHK // 16):
            d16 = idxD[2 + b, pl.ds(16 * k, 16)]
            li = d16 - wbase
            inw = (li >= 0) & (li < WINROWS)
            idxD[b, pl.ds(16 * k, 16)] = jnp.where(inw, li, WINROWS)
        return idxD.at[b]

    pltpu.sync_copy(src2.at[pl.ds(row_base, 2)], idxSA)
    pltpu.sync_copy(dst2.at[pl.ds(row_base, 2)], idxDA.at[pl.ds(2, 2)]
                    if wbase is not None else idxDA)

    def dst_slot(idxD):
        return idxD.at[pl.ds(2, 2)] if wbase is not None else idxD

    def it(j, carry):
        row0 = row_base + 4 * j
        cb1 = pltpu.async_copy(src2.at[pl.ds(row0 + 2, 2)], idxSB, semI)
        cb2 = pltpu.async_copy(dst2.at[pl.ds(row0 + 2, 2)], dst_slot(idxDB),
                               semI)
        livA0 = make_liv(idxDA, 0)
        livA1 = make_liv(idxDA, 1)
        g0 = pltpu.async_copy(feat_h.at[idxSA.at[0]], rowsA, semG0)
        g1 = pltpu.async_copy(feat_h.at[idxSA.at[1]], rowsB, semG1)
        g0.wait()
        s0 = pltpu.async_copy(rowsA, accum.at[livA0], semS0, add=True)
        g1.wait()
        s1 = pltpu.async_copy(rowsB, accum.at[livA1], semS1, add=True)
        cb1.wait()
        cb2.wait()
        s0.wait()
        s1.wait()
        rown = jnp.minimum(row0 + 4, total_rows - 2)
        ca1 = pltpu.async_copy(src2.at[pl.ds(rown, 2)], idxSA, semI)
        ca2 = pltpu.async_copy(dst2.at[pl.ds(rown, 2)], dst_slot(idxDA), semI)
        livB0 = make_liv(idxDB, 0)
        livB1 = make_liv(idxDB, 1)
        g2 = pltpu.async_copy(feat_h.at[idxSB.at[0]], rowsA, semG0)
        g3 = pltpu.async_copy(feat_h.at[idxSB.at[1]], rowsB, semG1)
        g2.wait()
        s2 = pltpu.async_copy(rowsA, accum.at[livB0], semS0, add=True)
        g3.wait()
        s3 = pltpu.async_copy(rowsB, accum.at[livB1], semS1, add=True)
        s2.wait()
        s3.wait()
        ca1.wait()
        ca2.wait()
        return carry

    lax.fori_loop(0, nit, it, 0)


def _sweep_scratch(idx_rows):
    return [
        pltpu.VMEM((CHK, 128), jnp.float32),
        pltpu.VMEM((CHK, 128), jnp.float32),
        pltpu.VMEM((2, CHK), jnp.int32),
        pltpu.VMEM((2, CHK), jnp.int32),
        pltpu.VMEM((idx_rows, CHK), jnp.int32),
        pltpu.VMEM((idx_rows, CHK), jnp.int32),
        pltpu.SemaphoreType.DMA,
        pltpu.SemaphoreType.DMA,
        pltpu.SemaphoreType.DMA,
        pltpu.SemaphoreType.DMA,
        pltpu.SemaphoreType.DMA,
        pltpu.SemaphoreType.DMA,
    ]


def _seg_table(feat, src2, dst2):
    rows_tot = src2.shape[0]
    nchk = rows_tot // (NC * NS)
    rpt = NACC_T // NS

    out_type = jax.ShapeDtypeStruct((NC, NACC_T, 128), jnp.float32)
    scratch = ([pltpu.VMEM_SHARED((NACC_T, 128), jnp.float32),
                pltpu.VMEM((ZB, 128), jnp.float32)]
               + _sweep_scratch(2))

    def body(feat_h, src_h, dst_h, out_h, accum, zbuf, *refs):
        semz = refs[-1]
        refs = refs[:-1]
        c = lax.axis_index("c")
        s = lax.axis_index("s")
        _fill_rows(zbuf, ZB, 128, 0.0)
        base = s * rpt
        _zero_stripe(zbuf, accum, base, rpt, semz)
        plsc.subcore_barrier()
        row_base = c * (rows_tot // 2) + s * nchk
        _pipelined_sweep(feat_h, src_h, dst_h, accum, s, row_base, nchk, refs)
        plsc.subcore_barrier()
        pltpu.sync_copy(accum.at[pl.ds(base, rpt)], out_h.at[c, pl.ds(base, rpt)])

    fn = pl.kernel(body, out_type=out_type, mesh=_MESH, scratch_types=scratch)
    return fn(feat, src2, dst2)


def _seg_win(feat, src2, dst2):
    rows_tot = src2.shape[0]
    nchk = rows_tot // NS
    rpt = NACC_W // NS

    out_type = jax.ShapeDtypeStruct((4, NACC_W, 128), jnp.float32)
    scratch = ([pltpu.VMEM_SHARED((NACC_W, 128), jnp.float32),
                pltpu.VMEM((ZB, 128), jnp.float32)]
               + _sweep_scratch(4))

    def body(feat_h, src_h, dst_h, out_h, accum, zbuf, *refs):
        semz = refs[-1]
        refs = refs[:-1]
        c = lax.axis_index("c")
        s = lax.axis_index("s")
        _fill_rows(zbuf, ZB, 128, 0.0)
        base = s * rpt
        row_base = s * nchk

        for w in range(2):
            wabs = c * 2 + w
            wbase = wabs * WINROWS
            _zero_stripe(zbuf, accum, base, rpt, refs[-1])
            plsc.subcore_barrier()
            _pipelined_sweep(feat_h, src_h, dst_h, accum, s, row_base, nchk,
                             refs, wbase=wbase)
            plsc.subcore_barrier()
            pltpu.sync_copy(accum.at[pl.ds(base, rpt)],
                            out_h.at[wabs, pl.ds(base, rpt)])
            plsc.subcore_barrier()

    fn = pl.kernel(body, out_type=out_type, mesh=_MESH, scratch_types=scratch)
    return fn(feat, src2, dst2)


def _tc_table_proj(x_table, wcat, qv2, wq, bq2):
    grid = (6,)

    def body(x_ref, w_ref, qv_ref, wq_ref, bq_ref, o1, o2, o3, o4):
        x = x_ref[...]
        acc = jnp.dot(x, w_ref[...], preferred_element_type=jnp.float32)
        o1[...] = acc[:, 0:128]
        o2[...] = acc[:, 128:256]
        o3[...] = acc[:, 256:384]
        qv = qv_ref[...]
        q = jnp.dot(qv, wq_ref[...], preferred_element_type=jnp.float32) + bq_ref[...]
        rown = jnp.sqrt(jnp.sum(x * x, axis=1, keepdims=True))
        nq = jnp.sqrt(jnp.sum(qv * qv))
        wts = jnp.maximum(
            acc[:, 384:385] / (jnp.maximum(rown, 1e-12) * jnp.maximum(nq, 1e-12)),
            0.0)
        o4[...] = wts * q

    ospec = pl.BlockSpec((BLK, 128), lambda i: (i, 0))
    return pl.pallas_call(
        body,
        grid=grid,
        in_specs=[
            pl.BlockSpec((BLK, 512), lambda i: (i, 0)),
            pl.BlockSpec((512, 512), lambda i: (0, 0)),
            pl.BlockSpec((1, 512), lambda i: (0, 0)),
            pl.BlockSpec((512, 128), lambda i: (0, 0)),
            pl.BlockSpec((1, 128), lambda i: (0, 0)),
        ],
        out_specs=[ospec] * 4,
        out_shape=[jax.ShapeDtypeStruct((N_TABLE, 128), jnp.float32)] * 4,
    )(x_table, wcat, qv2, wq, bq2)


def _tc_col_proj(x_column, wcat):
    grid = (28,)

    def body(x_ref, w_ref, o1, o2, o3):
        acc = jnp.dot(x_ref[...], w_ref[...], preferred_element_type=jnp.float32)
        o1[...] = acc[:, 0:128]
        o2[...] = acc[:, 128:256]
        o3[...] = acc[:, 256:384]

    ospec = pl.BlockSpec((BLK, 128), lambda i: (i, 0))
    return pl.pallas_call(
        body,
        grid=grid,
        in_specs=[
            pl.BlockSpec((BLK, 128), lambda i: (i, 0)),
            pl.BlockSpec((128, 384), lambda i: (0, 0)),
        ],
        out_specs=[ospec] * 3,
        out_shape=[jax.ShapeDtypeStruct((N_COL, 128), jnp.float32)] * 3,
    )(x_column, wcat)


def _tc_col1(sum_hc, cnt_hc, sum_cs, cnt_cs, xcr, bias, w):
    grid = (28,)

    def body(sh_ref, ch_ref, ss_ref, cs_ref, x_ref, b_ref, w_ref, out):
        chc = ch_ref[0, :, 0:1]
        ccs = cs_ref[0, :, 0:1]
        mhc = sh_ref[0] / jnp.maximum(chc, 1.0)
        mcs = ss_ref[0] / jnp.maximum(ccs, 1.0)
        col1 = jnp.maximum(mhc + mcs + x_ref[...] + b_ref[...], 0.0)
        out[...] = jnp.dot(col1, w_ref[...], preferred_element_type=jnp.float32)

    sspec = pl.BlockSpec((1, BLK, 128), lambda i: (i // 7, i % 7, 0))
    cspec = sspec
    return pl.pallas_call(
        body,
        grid=grid,
        in_specs=[
            sspec, cspec, sspec, cspec,
            pl.BlockSpec((BLK, 128), lambda i: (i, 0)),
            pl.BlockSpec((1, 128), lambda i: (0, 0)),
            pl.BlockSpec((128, 128), lambda i: (0, 0)),
        ],
        out_specs=pl.BlockSpec((BLK, 128), lambda i: (i, 0)),
        out_shape=jax.ShapeDtypeStruct((N_COL, 128), jnp.float32),
    )(sum_hc, cnt_hc, sum_cs, cnt_cs, xcr, bias, w)


def _tc_tab1(sum_rev, cnt_rev, sum_tt, cnt_tt, xtr, bias, wcat):
    grid = (6,)

    def body(sr_ref, cr_ref, st_ref, ct_ref, x_ref, b_ref, w_ref, o1, o2):
        cr = cr_ref[0, :, 0:1] + cr_ref[1, :, 0:1]
        ct = ct_ref[0, :, 0:1] + ct_ref[1, :, 0:1]
        mr = (sr_ref[0] + sr_ref[1]) / jnp.maximum(cr, 1.0)
        mt = (st_ref[0] + st_ref[1]) / jnp.maximum(ct, 1.0)
        tab1 = jnp.maximum(mr + mt + x_ref[...] + b_ref[...], 0.0)
        acc = jnp.dot(tab1, w_ref[...], preferred_element_type=jnp.float32)
        o1[...] = acc[:, 0:128]
        o2[...] = acc[:, 128:256]

    sspec = pl.BlockSpec((2, BLK, 128), lambda i: (0, i, 0))
    cspec = pl.BlockSpec((2, BLK, 128), lambda i: (0, i, 0))
    ospec = pl.BlockSpec((BLK, 128), lambda i: (i, 0))
    return pl.pallas_call(
        body,
        grid=grid,
        in_specs=[
            sspec, cspec, sspec, cspec,
            pl.BlockSpec((BLK, 128), lambda i: (i, 0)),
            pl.BlockSpec((1, 128), lambda i: (0, 0)),
            pl.BlockSpec((128, 256), lambda i: (0, 0)),
        ],
        out_specs=[ospec] * 2,
        out_shape=[jax.ShapeDtypeStruct((N_TABLE, 128), jnp.float32)] * 2,
    )(sum_rev, cnt_rev, sum_tt, cnt_tt, xtr, bias, wcat)


def _tc_x0(s2rev, cnt_rev, s2tt, cnt_tt, tab1r, z0, bias):
    grid = (6,)

    def body(sr_ref, cr_ref, st_ref, ct_ref, x_ref, z_ref, b_ref, oh, oy):
        cr = cr_ref[0, :, 0:1] + cr_ref[1, :, 0:1]
        ct = ct_ref[0, :, 0:1] + ct_ref[1, :, 0:1]
        mr = (sr_ref[0] + sr_ref[1]) / jnp.maximum(cr, 1.0)
        mt = (st_ref[0] + st_ref[1]) / jnp.maximum(ct, 1.0)
        x0 = mr + mt + x_ref[...] + b_ref[...] + z_ref[...]
        dinv = lax.rsqrt(ct + 1.0)
        oh[...] = x0
        oy[...] = dinv * x0

    sspec = pl.BlockSpec((2, BLK, 128), lambda i: (0, i, 0))
    cspec = pl.BlockSpec((2, BLK, 128), lambda i: (0, i, 0))
    ospec = pl.BlockSpec((BLK, 128), lambda i: (i, 0))
    return pl.pallas_call(
        body,
        grid=grid,
        in_specs=[
            sspec, cspec, sspec, cspec,
            pl.BlockSpec((BLK, 128), lambda i: (i, 0)),
            pl.BlockSpec((BLK, 128), lambda i: (i, 0)),
            pl.BlockSpec((1, 128), lambda i: (0, 0)),
        ],
        out_specs=[ospec] * 2,
        out_shape=[jax.ShapeDtypeStruct((N_TABLE, 128), jnp.float32)] * 2,
    )(s2rev, cnt_rev, s2tt, cnt_tt, tab1r, z0, bias)


def _tc_appnp_step(part, cnt_tt, y, h):
    grid = (6,)

    def body(p_ref, ct_ref, y_ref, h_ref, oz, oy):
        ct = ct_ref[0, :, 0:1] + ct_ref[1, :, 0:1]
        dinv = lax.rsqrt(ct + 1.0)
        psum = p_ref[0] + p_ref[1]
        z = (1.0 - ALPHA) * dinv * (psum + y_ref[...]) + ALPHA * h_ref[...]
        oz[...] = z
        oy[...] = dinv * z

    sspec = pl.BlockSpec((2, BLK, 128), lambda i: (0, i, 0))
    cspec = pl.BlockSpec((2, BLK, 128), lambda i: (0, i, 0))
    ospec = pl.BlockSpec((BLK, 128), lambda i: (i, 0))
    return pl.pallas_call(
        body,
        grid=grid,
        in_specs=[
            sspec, cspec,
            pl.BlockSpec((BLK, 128), lambda i: (i, 0)),
            pl.BlockSpec((BLK, 128), lambda i: (i, 0)),
        ],
        out_specs=[ospec] * 2,
        out_shape=[jax.ShapeDtypeStruct((N_TABLE, 128), jnp.float32)] * 2,
    )(part, cnt_tt, y, h)


def _tc_scores(z, wo_row, bo):
    grid = (6,)

    def body(z_ref, w_ref, b_ref, out):
        s = jnp.sum(z_ref[...] * w_ref[...], axis=1, keepdims=True) + b_ref[...]
        out[...] = jnp.broadcast_to(s, (BLK, 128))

    return pl.pallas_call(
        body,
        grid=grid,
        in_specs=[
            pl.BlockSpec((BLK, 128), lambda i: (i, 0)),
            pl.BlockSpec((1, 128), lambda i: (0, 0)),
            pl.BlockSpec((1, 1), lambda i: (0, 0)),
        ],
        out_specs=pl.BlockSpec((BLK, 128), lambda i: (i, 0)),
        out_shape=jax.ShapeDtypeStruct((N_TABLE, 128), jnp.float32),
    )(z, wo_row, bo)


def kernel(x_table, x_column, query_vec, hc_src, hc_dst, rev_src, rev_dst,
           cs_src, cs_dst, tt_src, tt_dst, params):
    p = params
    f32 = jnp.float32

    wcat1 = jnp.concatenate(
        [p["Wl1_hc"], p["Wl1_tt"], p["Wr1_rev"] + p["Wr1_tt"],
         query_vec[:, None], jnp.zeros((512, 127), f32)], axis=1)
    qv2 = query_vec[None, :]
    bq2 = p["bq"][None, :]
    o_hc, o_tt, xtr, z0 = _tc_table_proj(x_table, wcat1, qv2, p["Wq"], bq2)

    wcat2 = jnp.concatenate(
        [p["Wl1_cs"], p["Wl1_rev"], p["Wr1_hc"] + p["Wr1_cs"]], axis=1)
    o_cs, o_rev, xcr = _tc_col_proj(x_column, wcat2)

    hs, hd = _pad_edges(hc_src, hc_dst, N_COL, 4096)
    css, csd = _pad_edges(cs_src, cs_dst, N_COL, 4096)
    rs, rd = _pad_edges(rev_src, rev_dst, N_TABLE, 8192)
    ts, td = _pad_edges(tt_src, tt_dst, N_TABLE, 8192)

    onesf = jnp.ones((8, 128), f32)
    cnt_hc = _seg_win(onesf, jnp.zeros_like(hd), hd)
    cnt_cs = _seg_win(onesf, jnp.zeros_like(csd), csd)
    cnt_rev = _seg_table(onesf, jnp.zeros_like(rd), rd)
    cnt_tt = _seg_table(onesf, jnp.zeros_like(td), td)
    sum_hc = _seg_win(o_hc, hs, hd)
    sum_cs = _seg_win(o_cs, css, csd)
    sum_rev = _seg_table(o_rev, rs, rd)
    sum_tt = _seg_table(o_tt, ts, td)

    bias1c = (p["b1_hc"] + p["b1_cs"])[None, :]
    col1p = _tc_col1(sum_hc, cnt_hc, sum_cs, cnt_cs, xcr, bias1c, p["Wl2_rev"])

    bias1t = (p["b1_rev"] + p["b1_tt"])[None, :]
    wcat3 = jnp.concatenate([p["Wl2_tt"], p["Wr2_rev"] + p["Wr2_tt"]], axis=1)
    tab1p, tab1r = _tc_tab1(sum_rev, cnt_rev, sum_tt, cnt_tt, xtr, bias1t, wcat3)

    s2rev = _seg_table(col1p, rs, rd)
    s2tt = _seg_table(tab1p, ts, td)

    bias2t = (p["b2_rev"] + p["b2_tt"])[None, :]
    h, y = _tc_x0(s2rev, cnt_rev, s2tt, cnt_tt, tab1r, z0, bias2t)

    z = h
    for _ in range(K_APPNP):
        part = _seg_table(y, ts, td)
        z, y = _tc_appnp_step(part, cnt_tt, y, h)

    wo_row = p["Wo"].reshape(1, 128)
    bo = p["bo"].reshape(1, 1)
    sb = _tc_scores(z, wo_row, bo)
    return sb[:, 0]


# speedup vs baseline: 11.1831x; 10.9948x over previous
import jax
import jax.numpy as jnp
from jax import lax
from jax.experimental import pallas as pl
from jax.experimental.pallas import tpu as pltpu
from jax.experimental.pallas import tpu_sc as plsc

N_TABLE = 10000
N_COL = 50000
ALPHA = 0.2
K_APPNP = 10

NC, NS = 2, 16
CHK = 64
ZB = 32

NACC_T = 10752
WINROWS = 12544
NACC_W = 12800
BLK = 1792

_MESH = plsc.VectorSubcoreMesh(core_axis_name="c", subcore_axis_name="s")


def _pad_edges(src, dst, dump, mult):
    e = src.shape[0]
    epad = -(-e // mult) * mult
    if epad != e:
        pad = epad - e
        src = jnp.concatenate([src, jnp.zeros((pad,), jnp.int32)])
        dst = jnp.concatenate([dst, jnp.full((pad,), dump, jnp.int32)])
    return src.reshape(-1, CHK), dst.reshape(-1, CHK)


def _fill_rows(ref, nrows, ncols, value):
    v16 = jnp.full((16,), value, jnp.float32)

    def row(r, carry):
        for k in range(ncols // 16):
            ref[r, pl.ds(16 * k, 16)] = v16
        return carry

    lax.fori_loop(0, nrows, row, 0)


def _zero_stripe(zbuf, ref, base, nrows, semz):
    cps = []
    off = 0
    while off < nrows:
        sz = min(ZB, nrows - off)
        cps.append(pltpu.async_copy(zbuf.at[pl.ds(0, sz)],
                                    ref.at[pl.ds(base + off, sz)], semz))
        off += sz
    for cp in cps:
        cp.wait()


def _pipelined_sweep(feat_h, src2, dst2, accum, s, row_base, nchk, refs,
                     wbase=None):
    (rowsA, rowsB, idxSA, idxSB, idxDA, idxDB,
     semI, semG0, semG1, semS0, semS1) = refs
    nit = nchk // 4
    total_rows = src2.shape[0]

    def make_liv(idxD, b):
        if wbase is None:
            return idxD.at[b]
        for k in range(CHK // 16):
            d16 = idxD[2 + b, pl.ds(16 * k, 16)]
            li = d16 - wbase
            inw = (li >= 0) & (li < WINROWS)
            idxD[b, pl.ds(16 * k, 16)] = jnp.where(inw, li, WINROWS)
        return idxD.at[b]

    pltpu.sync_copy(src2.at[pl.ds(row_base, 2)], idxSA)
    pltpu.sync_copy(dst2.at[pl.ds(row_base, 2)], idxDA.at[pl.ds(2, 2)]
                    if wbase is not None else idxDA)

    def dst_slot(idxD):
        return idxD.at[pl.ds(2, 2)] if wbase is not None else idxD

    def it(j, carry):
        row0 = row_base + 4 * j
        cb1 = pltpu.async_copy(src2.at[pl.ds(row0 + 2, 2)], idxSB, semI)
        cb2 = pltpu.async_copy(dst2.at[pl.ds(row0 + 2, 2)], dst_slot(idxDB),
                               semI)
        livA0 = make_liv(idxDA, 0)
        livA1 = make_liv(idxDA, 1)
        g0 = pltpu.async_copy(feat_h.at[idxSA.at[0]], rowsA, semG0)
        g1 = pltpu.async_copy(feat_h.at[idxSA.at[1]], rowsB, semG1)
        g0.wait()
        s0 = pltpu.async_copy(rowsA, accum.at[livA0], semS0, add=True)
        g1.wait()
        s1 = pltpu.async_copy(rowsB, accum.at[livA1], semS1, add=True)
        cb1.wait()
        cb2.wait()
        s0.wait()
        s1.wait()
        rown = jnp.minimum(row0 + 4, total_rows - 2)
        ca1 = pltpu.async_copy(src2.at[pl.ds(rown, 2)], idxSA, semI)
        ca2 = pltpu.async_copy(dst2.at[pl.ds(rown, 2)], dst_slot(idxDA), semI)
        livB0 = make_liv(idxDB, 0)
        livB1 = make_liv(idxDB, 1)
        g2 = pltpu.async_copy(feat_h.at[idxSB.at[0]], rowsA, semG0)
        g3 = pltpu.async_copy(feat_h.at[idxSB.at[1]], rowsB, semG1)
        g2.wait()
        s2 = pltpu.async_copy(rowsA, accum.at[livB0], semS0, add=True)
        g3.wait()
        s3 = pltpu.async_copy(rowsB, accum.at[livB1], semS1, add=True)
        s2.wait()
        s3.wait()
        ca1.wait()
        ca2.wait()
        return carry

    lax.fori_loop(0, nit, it, 0)


def _sweep_scratch(idx_rows):
    return [
        pltpu.VMEM((CHK, 128), jnp.float32),
        pltpu.VMEM((CHK, 128), jnp.float32),
        pltpu.VMEM((2, CHK), jnp.int32),
        pltpu.VMEM((2, CHK), jnp.int32),
        pltpu.VMEM((idx_rows, CHK), jnp.int32),
        pltpu.VMEM((idx_rows, CHK), jnp.int32),
        pltpu.SemaphoreType.DMA,
        pltpu.SemaphoreType.DMA,
        pltpu.SemaphoreType.DMA,
        pltpu.SemaphoreType.DMA,
        pltpu.SemaphoreType.DMA,
        pltpu.SemaphoreType.DMA,
    ]


def _seg_table(feat, src2, dst2):
    rows_tot = src2.shape[0]
    nchk = rows_tot // (NC * NS)
    rpt = NACC_T // NS

    out_type = jax.ShapeDtypeStruct((NC, NACC_T, 128), jnp.float32)
    scratch = ([pltpu.VMEM_SHARED((NACC_T, 128), jnp.float32),
                pltpu.VMEM((ZB, 128), jnp.float32)]
               + _sweep_scratch(2))

    def body(feat_h, src_h, dst_h, out_h, accum, zbuf, *refs):
        semz = refs[-1]
        refs = refs[:-1]
        c = lax.axis_index("c")
        s = lax.axis_index("s")
        _fill_rows(zbuf, ZB, 128, 0.0)
        base = s * rpt
        _zero_stripe(zbuf, accum, base, rpt, semz)
        plsc.subcore_barrier()
        row_base = c * (rows_tot // 2) + s * nchk
        _pipelined_sweep(feat_h, src_h, dst_h, accum, s, row_base, nchk, refs)
        plsc.subcore_barrier()
        pltpu.sync_copy(accum.at[pl.ds(base, rpt)], out_h.at[c, pl.ds(base, rpt)])

    fn = pl.kernel(body, out_type=out_type, mesh=_MESH, scratch_types=scratch)
    return fn(feat, src2, dst2)


def _seg_win(feat, src2, dst2):
    rows_tot = src2.shape[0]
    nchk = rows_tot // NS
    rpt = NACC_W // NS

    out_type = jax.ShapeDtypeStruct((4, NACC_W, 128), jnp.float32)
    scratch = ([pltpu.VMEM_SHARED((NACC_W, 128), jnp.float32),
                pltpu.VMEM((ZB, 128), jnp.float32)]
               + _sweep_scratch(4))

    def body(feat_h, src_h, dst_h, out_h, accum, zbuf, *refs):
        semz = refs[-1]
        refs = refs[:-1]
        c = lax.axis_index("c")
        s = lax.axis_index("s")
        _fill_rows(zbuf, ZB, 128, 0.0)
        base = s * rpt
        row_base = s * nchk

        for w in range(2):
            wabs = c * 2 + w
            wbase = wabs * WINROWS
            _zero_stripe(zbuf, accum, base, rpt, refs[-1])
            plsc.subcore_barrier()
            _pipelined_sweep(feat_h, src_h, dst_h, accum, s, row_base, nchk,
                             refs, wbase=wbase)
            plsc.subcore_barrier()
            pltpu.sync_copy(accum.at[pl.ds(base, rpt)],
                            out_h.at[wabs, pl.ds(base, rpt)])
            plsc.subcore_barrier()

    fn = pl.kernel(body, out_type=out_type, mesh=_MESH, scratch_types=scratch)
    return fn(feat, src2, dst2)


def _tc_table_proj(x_table, wcat, qv2, wq, bq2):
    grid = (6,)

    def body(x_ref, w_ref, qv_ref, wq_ref, bq_ref, o1, o2, o3, o4):
        x = x_ref[...]
        acc = jnp.dot(x, w_ref[...], preferred_element_type=jnp.float32)
        o1[...] = acc[:, 0:128]
        o2[...] = acc[:, 128:256]
        o3[...] = acc[:, 256:384]
        qv = qv_ref[...]
        q = jnp.dot(qv, wq_ref[...], preferred_element_type=jnp.float32) + bq_ref[...]
        rown = jnp.sqrt(jnp.sum(x * x, axis=1, keepdims=True))
        nq = jnp.sqrt(jnp.sum(qv * qv))
        wts = jnp.maximum(
            acc[:, 384:385] / (jnp.maximum(rown, 1e-12) * jnp.maximum(nq, 1e-12)),
            0.0)
        o4[...] = wts * q

    ospec = pl.BlockSpec((BLK, 128), lambda i: (i, 0))
    return pl.pallas_call(
        body,
        grid=grid,
        in_specs=[
            pl.BlockSpec((BLK, 512), lambda i: (i, 0)),
            pl.BlockSpec((512, 512), lambda i: (0, 0)),
            pl.BlockSpec((1, 512), lambda i: (0, 0)),
            pl.BlockSpec((512, 128), lambda i: (0, 0)),
            pl.BlockSpec((1, 128), lambda i: (0, 0)),
        ],
        out_specs=[ospec] * 4,
        out_shape=[jax.ShapeDtypeStruct((N_TABLE, 128), jnp.float32)] * 4,
    )(x_table, wcat, qv2, wq, bq2)


def _tc_col_proj(x_column, wcat):
    grid = (28,)

    def body(x_ref, w_ref, o1, o2, o3):
        acc = jnp.dot(x_ref[...], w_ref[...], preferred_element_type=jnp.float32)
        o1[...] = acc[:, 0:128]
        o2[...] = acc[:, 128:256]
        o3[...] = acc[:, 256:384]

    ospec = pl.BlockSpec((BLK, 128), lambda i: (i, 0))
    return pl.pallas_call(
        body,
        grid=grid,
        in_specs=[
            pl.BlockSpec((BLK, 128), lambda i: (i, 0)),
            pl.BlockSpec((128, 384), lambda i: (0, 0)),
        ],
        out_specs=[ospec] * 3,
        out_shape=[jax.ShapeDtypeStruct((N_COL, 128), jnp.float32)] * 3,
    )(x_column, wcat)


def _tc_col1(sum_hc, cnt_hc, sum_cs, cnt_cs, xcr, bias, w):
    grid = (28,)

    def body(sh_ref, ch_ref, ss_ref, cs_ref, x_ref, b_ref, w_ref, out):
        chc = ch_ref[0, :, 0:1]
        ccs = cs_ref[0, :, 0:1]
        mhc = sh_ref[0] / jnp.maximum(chc, 1.0)
        mcs = ss_ref[0] / jnp.maximum(ccs, 1.0)
        col1 = jnp.maximum(mhc + mcs + x_ref[...] + b_ref[...], 0.0)
        out[...] = jnp.dot(col1, w_ref[...], preferred_element_type=jnp.float32)

    sspec = pl.BlockSpec((1, BLK, 128), lambda i: (i // 7, i % 7, 0))
    cspec = sspec
    return pl.pallas_call(
        body,
        grid=grid,
        in_specs=[
            sspec, cspec, sspec, cspec,
            pl.BlockSpec((BLK, 128), lambda i: (i, 0)),
            pl.BlockSpec((1, 128), lambda i: (0, 0)),
            pl.BlockSpec((128, 128), lambda i: (0, 0)),
        ],
        out_specs=pl.BlockSpec((BLK, 128), lambda i: (i, 0)),
        out_shape=jax.ShapeDtypeStruct((N_COL, 128), jnp.float32),
    )(sum_hc, cnt_hc, sum_cs, cnt_cs, xcr, bias, w)


def _tc_tab1(sum_rev, cnt_rev, sum_tt, cnt_tt, xtr, bias, wcat):
    grid = (6,)

    def body(sr_ref, cr_ref, st_ref, ct_ref, x_ref, b_ref, w_ref, o1, o2):
        cr = cr_ref[0, :, 0:1] + cr_ref[1, :, 0:1]
        ct = ct_ref[0, :, 0:1] + ct_ref[1, :, 0:1]
        mr = (sr_ref[0] + sr_ref[1]) / jnp.maximum(cr, 1.0)
        mt = (st_ref[0] + st_ref[1]) / jnp.maximum(ct, 1.0)
        tab1 = jnp.maximum(mr + mt + x_ref[...] + b_ref[...], 0.0)
        acc = jnp.dot(tab1, w_ref[...], preferred_element_type=jnp.float32)
        o1[...] = acc[:, 0:128]
        o2[...] = acc[:, 128:256]

    sspec = pl.BlockSpec((2, BLK, 128), lambda i: (0, i, 0))
    cspec = pl.BlockSpec((2, BLK, 128), lambda i: (0, i, 0))
    ospec = pl.BlockSpec((BLK, 128), lambda i: (i, 0))
    return pl.pallas_call(
        body,
        grid=grid,
        in_specs=[
            sspec, cspec, sspec, cspec,
            pl.BlockSpec((BLK, 128), lambda i: (i, 0)),
            pl.BlockSpec((1, 128), lambda i: (0, 0)),
            pl.BlockSpec((128, 256), lambda i: (0, 0)),
        ],
        out_specs=[ospec] * 2,
        out_shape=[jax.ShapeDtypeStruct((N_TABLE, 128), jnp.float32)] * 2,
    )(sum_rev, cnt_rev, sum_tt, cnt_tt, xtr, bias, wcat)


def _tc_x0(s2rev, cnt_rev, s2tt, cnt_tt, tab1r, z0, bias):
    grid = (6,)

    def body(sr_ref, cr_ref, st_ref, ct_ref, x_ref, z_ref, b_ref, oh, oy):
        cr = cr_ref[0, :, 0:1] + cr_ref[1, :, 0:1]
        ct = ct_ref[0, :, 0:1] + ct_ref[1, :, 0:1]
        mr = (sr_ref[0] + sr_ref[1]) / jnp.maximum(cr, 1.0)
        mt = (st_ref[0] + st_ref[1]) / jnp.maximum(ct, 1.0)
        x0 = mr + mt + x_ref[...] + b_ref[...] + z_ref[...]
        dinv = lax.rsqrt(ct + 1.0)
        oh[...] = x0
        oy[...] = dinv * x0

    sspec = pl.BlockSpec((2, BLK, 128), lambda i: (0, i, 0))
    cspec = pl.BlockSpec((2, BLK, 128), lambda i: (0, i, 0))
    ospec = pl.BlockSpec((BLK, 128), lambda i: (i, 0))
    return pl.pallas_call(
        body,
        grid=grid,
        in_specs=[
            sspec, cspec, sspec, cspec,
            pl.BlockSpec((BLK, 128), lambda i: (i, 0)),
            pl.BlockSpec((BLK, 128), lambda i: (i, 0)),
            pl.BlockSpec((1, 128), lambda i: (0, 0)),
        ],
        out_specs=[ospec] * 2,
        out_shape=[jax.ShapeDtypeStruct((N_TABLE, 128), jnp.float32)] * 2,
    )(s2rev, cnt_rev, s2tt, cnt_tt, tab1r, z0, bias)


def _tc_appnp_step(part, cnt_tt, y, h):
    grid = (6,)

    def body(p_ref, ct_ref, y_ref, h_ref, oz, oy):
        ct = ct_ref[0, :, 0:1] + ct_ref[1, :, 0:1]
        dinv = lax.rsqrt(ct + 1.0)
        psum = p_ref[0] + p_ref[1]
        z = (1.0 - ALPHA) * dinv * (psum + y_ref[...]) + ALPHA * h_ref[...]
        oz[...] = z
        oy[...] = dinv * z

    sspec = pl.BlockSpec((2, BLK, 128), lambda i: (0, i, 0))
    cspec = pl.BlockSpec((2, BLK, 128), lambda i: (0, i, 0))
    ospec = pl.BlockSpec((BLK, 128), lambda i: (i, 0))
    return pl.pallas_call(
        body,
        grid=grid,
        in_specs=[
            sspec, cspec,
            pl.BlockSpec((BLK, 128), lambda i: (i, 0)),
            pl.BlockSpec((BLK, 128), lambda i: (i, 0)),
        ],
        out_specs=[ospec] * 2,
        out_shape=[jax.ShapeDtypeStruct((N_TABLE, 128), jnp.float32)] * 2,
    )(part, cnt_tt, y, h)


def _tc_scores(z, wo_row, bo):
    grid = (6,)

    def body(z_ref, w_ref, b_ref, out):
        s = jnp.sum(z_ref[...] * w_ref[...], axis=1, keepdims=True) + b_ref[...]
        out[...] = jnp.broadcast_to(s, (BLK, 128))

    return pl.pallas_call(
        body,
        grid=grid,
        in_specs=[
            pl.BlockSpec((BLK, 128), lambda i: (i, 0)),
            pl.BlockSpec((1, 128), lambda i: (0, 0)),
            pl.BlockSpec((1, 1), lambda i: (0, 0)),
        ],
        out_specs=pl.BlockSpec((BLK, 128), lambda i: (i, 0)),
        out_shape=jax.ShapeDtypeStruct((N_TABLE, 128), jnp.float32),
    )(z, wo_row, bo)


def kernel(x_table, x_column, query_vec, hc_src, hc_dst, rev_src, rev_dst,
           cs_src, cs_dst, tt_src, tt_dst, params):
    p = params
    f32 = jnp.float32

    wcat1 = jnp.concatenate(
        [p["Wl1_hc"], p["Wl1_tt"], p["Wr1_rev"] + p["Wr1_tt"],
         query_vec[:, None], jnp.zeros((512, 127), f32)], axis=1)
    qv2 = query_vec[None, :]
    bq2 = p["bq"][None, :]
    o_hc, o_tt, xtr, z0 = _tc_table_proj(x_table, wcat1, qv2, p["Wq"], bq2)

    wcat2 = jnp.concatenate(
        [p["Wl1_cs"], p["Wl1_rev"], p["Wr1_hc"] + p["Wr1_cs"]], axis=1)
    o_cs, o_rev, xcr = _tc_col_proj(x_column, wcat2)

    hs, hd = _pad_edges(hc_src, hc_dst, N_COL, 4096)
    css, csd = _pad_edges(cs_src, cs_dst, N_COL, 4096)
    rs, rd = _pad_edges(rev_src, rev_dst, N_TABLE, 8192)
    ts, td = _pad_edges(tt_src, tt_dst, N_TABLE, 8192)

    onesf = jnp.ones((N_COL + 64, 128), f32)
    cnt_hc = _seg_win(onesf, hs, hd)
    cnt_cs = _seg_win(onesf, css, csd)
    cnt_rev = _seg_table(onesf, rs, rd)
    cnt_tt = _seg_table(onesf, ts, td)
    sum_hc = _seg_win(o_hc, hs, hd)
    sum_cs = _seg_win(o_cs, css, csd)
    sum_rev = _seg_table(o_rev, rs, rd)
    sum_tt = _seg_table(o_tt, ts, td)

    bias1c = (p["b1_hc"] + p["b1_cs"])[None, :]
    col1p = _tc_col1(sum_hc, cnt_hc, sum_cs, cnt_cs, xcr, bias1c, p["Wl2_rev"])

    bias1t = (p["b1_rev"] + p["b1_tt"])[None, :]
    wcat3 = jnp.concatenate([p["Wl2_tt"], p["Wr2_rev"] + p["Wr2_tt"]], axis=1)
    tab1p, tab1r = _tc_tab1(sum_rev, cnt_rev, sum_tt, cnt_tt, xtr, bias1t, wcat3)

    s2rev = _seg_table(col1p, rs, rd)
    s2tt = _seg_table(tab1p, ts, td)

    bias2t = (p["b2_rev"] + p["b2_tt"])[None, :]
    h, y = _tc_x0(s2rev, cnt_rev, s2tt, cnt_tt, tab1r, z0, bias2t)

    z = h
    for _ in range(K_APPNP):
        part = _seg_table(y, ts, td)
        z, y = _tc_appnp_step(part, cnt_tt, y, h)

    wo_row = p["Wo"].reshape(1, 128)
    bo = p["bo"].reshape(1, 1)
    sb = _tc_scores(z, wo_row, bo)
    return sb[:, 0]


# speedup vs baseline: 13.8687x; 1.2401x over previous
import jax
import jax.numpy as jnp
from jax import lax
from jax.experimental import pallas as pl
from jax.experimental.pallas import tpu as pltpu
from jax.experimental.pallas import tpu_sc as plsc

N_TABLE = 10000
N_COL = 50000
ALPHA = 0.2
K_APPNP = 10

NC, NS = 2, 16
CHK = 64
ZB = 32

NACC_T = 10752
WINROWS = 12544
NACC_W = 12800
BLK = 1792

_MESH = plsc.VectorSubcoreMesh(core_axis_name="c", subcore_axis_name="s")


def _pad_edges(src, dst, dump, mult):
    e = src.shape[0]
    epad = -(-e // mult) * mult
    if epad != e:
        pad = epad - e
        src = jnp.concatenate([src, jnp.zeros((pad,), jnp.int32)])
        dst = jnp.concatenate([dst, jnp.full((pad,), dump, jnp.int32)])
    return src.reshape(-1, CHK), dst.reshape(-1, CHK)


def _fill_rows(ref, nrows, ncols, value):
    v16 = jnp.full((16,), value, jnp.float32)

    def row(r, carry):
        for k in range(ncols // 16):
            ref[r, pl.ds(16 * k, 16)] = v16
        return carry

    lax.fori_loop(0, nrows, row, 0)


def _zero_stripe(zbuf, ref, base, nrows, semz):
    cps = []
    off = 0
    while off < nrows:
        sz = min(ZB, nrows - off)
        cps.append(pltpu.async_copy(zbuf.at[pl.ds(0, sz)],
                                    ref.at[pl.ds(base + off, sz)], semz))
        off += sz
    for cp in cps:
        cp.wait()


def _pipelined_sweep(feat_h, src2, dst2, accum, s, row_base, nchk, refs,
                     wbase=None, ones_mode=False):
    (rowsA, rowsB, idxSA, idxSB, idxDA, idxDB,
     semI, semG0, semG1, semS0, semS1) = refs
    nit = nchk // 4
    total_rows = src2.shape[0]

    def make_liv(idxD, b):
        if wbase is None:
            return idxD.at[b]
        for k in range(CHK // 16):
            d16 = idxD[2 + b, pl.ds(16 * k, 16)]
            li = d16 - wbase
            inw = (li >= 0) & (li < WINROWS)
            idxD[b, pl.ds(16 * k, 16)] = jnp.where(inw, li, WINROWS)
        return idxD.at[b]

    pltpu.sync_copy(src2.at[pl.ds(row_base, 2)], idxSA)
    pltpu.sync_copy(dst2.at[pl.ds(row_base, 2)], idxDA.at[pl.ds(2, 2)]
                    if wbase is not None else idxDA)

    def dst_slot(idxD):
        return idxD.at[pl.ds(2, 2)] if wbase is not None else idxD

    def it(j, carry):
        row0 = row_base + 4 * j
        cb1 = pltpu.async_copy(src2.at[pl.ds(row0 + 2, 2)], idxSB, semI)
        cb2 = pltpu.async_copy(dst2.at[pl.ds(row0 + 2, 2)], dst_slot(idxDB),
                               semI)
        livA0 = make_liv(idxDA, 0)
        livA1 = make_liv(idxDA, 1)
        if not ones_mode:
            g0 = pltpu.async_copy(feat_h.at[idxSA.at[0]], rowsA, semG0)
            g1 = pltpu.async_copy(feat_h.at[idxSA.at[1]], rowsB, semG1)
            g0.wait()
        s0 = pltpu.async_copy(rowsA, accum.at[livA0], semS0, add=True)
        if not ones_mode:
            g1.wait()
        s1 = pltpu.async_copy(rowsB, accum.at[livA1], semS1, add=True)
        cb1.wait()
        cb2.wait()
        s0.wait()
        s1.wait()
        rown = jnp.minimum(row0 + 4, total_rows - 2)
        ca1 = pltpu.async_copy(src2.at[pl.ds(rown, 2)], idxSA, semI)
        ca2 = pltpu.async_copy(dst2.at[pl.ds(rown, 2)], dst_slot(idxDA), semI)
        livB0 = make_liv(idxDB, 0)
        livB1 = make_liv(idxDB, 1)
        if not ones_mode:
            g2 = pltpu.async_copy(feat_h.at[idxSB.at[0]], rowsA, semG0)
            g3 = pltpu.async_copy(feat_h.at[idxSB.at[1]], rowsB, semG1)
            g2.wait()
        s2 = pltpu.async_copy(rowsA, accum.at[livB0], semS0, add=True)
        if not ones_mode:
            g3.wait()
        s3 = pltpu.async_copy(rowsB, accum.at[livB1], semS1, add=True)
        s2.wait()
        s3.wait()
        ca1.wait()
        ca2.wait()
        return carry

    lax.fori_loop(0, nit, it, 0)


def _sweep_scratch(idx_rows):
    return [
        pltpu.VMEM((CHK, 128), jnp.float32),
        pltpu.VMEM((CHK, 128), jnp.float32),
        pltpu.VMEM((2, CHK), jnp.int32),
        pltpu.VMEM((2, CHK), jnp.int32),
        pltpu.VMEM((idx_rows, CHK), jnp.int32),
        pltpu.VMEM((idx_rows, CHK), jnp.int32),
        pltpu.SemaphoreType.DMA,
        pltpu.SemaphoreType.DMA,
        pltpu.SemaphoreType.DMA,
        pltpu.SemaphoreType.DMA,
        pltpu.SemaphoreType.DMA,
        pltpu.SemaphoreType.DMA,
    ]


def _seg_table(feat, src2, dst2, ones_mode=False):
    rows_tot = src2.shape[0]
    nchk = rows_tot // (NC * NS)
    rpt = NACC_T // NS

    out_type = jax.ShapeDtypeStruct((NC, NACC_T, 128), jnp.float32)
    scratch = ([pltpu.VMEM_SHARED((NACC_T, 128), jnp.float32),
                pltpu.VMEM((ZB, 128), jnp.float32)]
               + _sweep_scratch(2))

    def body(feat_h, src_h, dst_h, out_h, accum, zbuf, *refs):
        semz = refs[-1]
        refs = refs[:-1]
        c = lax.axis_index("c")
        s = lax.axis_index("s")
        _fill_rows(zbuf, ZB, 128, 0.0)
        if ones_mode:
            _fill_rows(refs[0], CHK, 128, 1.0)
            _fill_rows(refs[1], CHK, 128, 1.0)
        base = s * rpt
        _zero_stripe(zbuf, accum, base, rpt, semz)
        plsc.subcore_barrier()
        row_base = c * (rows_tot // 2) + s * nchk
        _pipelined_sweep(feat_h, src_h, dst_h, accum, s, row_base, nchk, refs,
                         ones_mode=ones_mode)
        plsc.subcore_barrier()
        pltpu.sync_copy(accum.at[pl.ds(base, rpt)], out_h.at[c, pl.ds(base, rpt)])

    fn = pl.kernel(body, out_type=out_type, mesh=_MESH, scratch_types=scratch)
    return fn(feat, src2, dst2)


def _seg_win(feat, src2, dst2, ones_mode=False):
    rows_tot = src2.shape[0]
    nchk = rows_tot // NS
    rpt = NACC_W // NS

    out_type = jax.ShapeDtypeStruct((4, NACC_W, 128), jnp.float32)
    scratch = ([pltpu.VMEM_SHARED((NACC_W, 128), jnp.float32),
                pltpu.VMEM((ZB, 128), jnp.float32)]
               + _sweep_scratch(4))

    def body(feat_h, src_h, dst_h, out_h, accum, zbuf, *refs):
        semz = refs[-1]
        refs = refs[:-1]
        c = lax.axis_index("c")
        s = lax.axis_index("s")
        _fill_rows(zbuf, ZB, 128, 0.0)
        if ones_mode:
            _fill_rows(refs[0], CHK, 128, 1.0)
            _fill_rows(refs[1], CHK, 128, 1.0)
        base = s * rpt
        row_base = s * nchk

        for w in range(2):
            wabs = c * 2 + w
            wbase = wabs * WINROWS
            _zero_stripe(zbuf, accum, base, rpt, refs[-1])
            plsc.subcore_barrier()
            _pipelined_sweep(feat_h, src_h, dst_h, accum, s, row_base, nchk,
                             refs, wbase=wbase, ones_mode=ones_mode)
            plsc.subcore_barrier()
            pltpu.sync_copy(accum.at[pl.ds(base, rpt)],
                            out_h.at[wabs, pl.ds(base, rpt)])
            plsc.subcore_barrier()

    fn = pl.kernel(body, out_type=out_type, mesh=_MESH, scratch_types=scratch)
    return fn(feat, src2, dst2)


def _tc_table_proj(x_table, wcat, qv2, wq, bq2):
    grid = (6,)

    def body(x_ref, w_ref, qv_ref, wq_ref, bq_ref, o1, o2, o3, o4):
        x = x_ref[...]
        acc = jnp.dot(x, w_ref[...], preferred_element_type=jnp.float32)
        o1[...] = acc[:, 0:128]
        o2[...] = acc[:, 128:256]
        o3[...] = acc[:, 256:384]
        qv = qv_ref[...]
        q = jnp.dot(qv, wq_ref[...], preferred_element_type=jnp.float32) + bq_ref[...]
        rown = jnp.sqrt(jnp.sum(x * x, axis=1, keepdims=True))
        nq = jnp.sqrt(jnp.sum(qv * qv))
        wts = jnp.maximum(
            acc[:, 384:385] / (jnp.maximum(rown, 1e-12) * jnp.maximum(nq, 1e-12)),
            0.0)
        o4[...] = wts * q

    ospec = pl.BlockSpec((BLK, 128), lambda i: (i, 0))
    return pl.pallas_call(
        body,
        grid=grid,
        in_specs=[
            pl.BlockSpec((BLK, 512), lambda i: (i, 0)),
            pl.BlockSpec((512, 512), lambda i: (0, 0)),
            pl.BlockSpec((1, 512), lambda i: (0, 0)),
            pl.BlockSpec((512, 128), lambda i: (0, 0)),
            pl.BlockSpec((1, 128), lambda i: (0, 0)),
        ],
        out_specs=[ospec] * 4,
        out_shape=[jax.ShapeDtypeStruct((N_TABLE, 128), jnp.float32)] * 4,
    )(x_table, wcat, qv2, wq, bq2)


def _tc_col_proj(x_column, wcat):
    grid = (28,)

    def body(x_ref, w_ref, o1, o2, o3):
        acc = jnp.dot(x_ref[...], w_ref[...], preferred_element_type=jnp.float32)
        o1[...] = acc[:, 0:128]
        o2[...] = acc[:, 128:256]
        o3[...] = acc[:, 256:384]

    ospec = pl.BlockSpec((BLK, 128), lambda i: (i, 0))
    return pl.pallas_call(
        body,
        grid=grid,
        in_specs=[
            pl.BlockSpec((BLK, 128), lambda i: (i, 0)),
            pl.BlockSpec((128, 384), lambda i: (0, 0)),
        ],
        out_specs=[ospec] * 3,
        out_shape=[jax.ShapeDtypeStruct((N_COL, 128), jnp.float32)] * 3,
    )(x_column, wcat)


def _tc_col1(sum_hc, cnt_hc, sum_cs, cnt_cs, xcr, bias, w):
    grid = (28,)

    def body(sh_ref, ch_ref, ss_ref, cs_ref, x_ref, b_ref, w_ref, out):
        chc = ch_ref[0, :, 0:1]
        ccs = cs_ref[0, :, 0:1]
        mhc = sh_ref[0] / jnp.maximum(chc, 1.0)
        mcs = ss_ref[0] / jnp.maximum(ccs, 1.0)
        col1 = jnp.maximum(mhc + mcs + x_ref[...] + b_ref[...], 0.0)
        out[...] = jnp.dot(col1, w_ref[...], preferred_element_type=jnp.float32)

    sspec = pl.BlockSpec((1, BLK, 128), lambda i: (i // 7, i % 7, 0))
    cspec = sspec
    return pl.pallas_call(
        body,
        grid=grid,
        in_specs=[
            sspec, cspec, sspec, cspec,
            pl.BlockSpec((BLK, 128), lambda i: (i, 0)),
            pl.BlockSpec((1, 128), lambda i: (0, 0)),
            pl.BlockSpec((128, 128), lambda i: (0, 0)),
        ],
        out_specs=pl.BlockSpec((BLK, 128), lambda i: (i, 0)),
        out_shape=jax.ShapeDtypeStruct((N_COL, 128), jnp.float32),
    )(sum_hc, cnt_hc, sum_cs, cnt_cs, xcr, bias, w)


def _tc_tab1(sum_rev, cnt_rev, sum_tt, cnt_tt, xtr, bias, wcat):
    grid = (6,)

    def body(sr_ref, cr_ref, st_ref, ct_ref, x_ref, b_ref, w_ref, o1, o2):
        cr = cr_ref[0, :, 0:1] + cr_ref[1, :, 0:1]
        ct = ct_ref[0, :, 0:1] + ct_ref[1, :, 0:1]
        mr = (sr_ref[0] + sr_ref[1]) / jnp.maximum(cr, 1.0)
        mt = (st_ref[0] + st_ref[1]) / jnp.maximum(ct, 1.0)
        tab1 = jnp.maximum(mr + mt + x_ref[...] + b_ref[...], 0.0)
        acc = jnp.dot(tab1, w_ref[...], preferred_element_type=jnp.float32)
        o1[...] = acc[:, 0:128]
        o2[...] = acc[:, 128:256]

    sspec = pl.BlockSpec((2, BLK, 128), lambda i: (0, i, 0))
    cspec = pl.BlockSpec((2, BLK, 128), lambda i: (0, i, 0))
    ospec = pl.BlockSpec((BLK, 128), lambda i: (i, 0))
    return pl.pallas_call(
        body,
        grid=grid,
        in_specs=[
            sspec, cspec, sspec, cspec,
            pl.BlockSpec((BLK, 128), lambda i: (i, 0)),
            pl.BlockSpec((1, 128), lambda i: (0, 0)),
            pl.BlockSpec((128, 256), lambda i: (0, 0)),
        ],
        out_specs=[ospec] * 2,
        out_shape=[jax.ShapeDtypeStruct((N_TABLE, 128), jnp.float32)] * 2,
    )(sum_rev, cnt_rev, sum_tt, cnt_tt, xtr, bias, wcat)


def _tc_x0(s2rev, cnt_rev, s2tt, cnt_tt, tab1r, z0, bias):
    grid = (6,)

    def body(sr_ref, cr_ref, st_ref, ct_ref, x_ref, z_ref, b_ref, oh, oy):
        cr = cr_ref[0, :, 0:1] + cr_ref[1, :, 0:1]
        ct = ct_ref[0, :, 0:1] + ct_ref[1, :, 0:1]
        mr = (sr_ref[0] + sr_ref[1]) / jnp.maximum(cr, 1.0)
        mt = (st_ref[0] + st_ref[1]) / jnp.maximum(ct, 1.0)
        x0 = mr + mt + x_ref[...] + b_ref[...] + z_ref[...]
        dinv = lax.rsqrt(ct + 1.0)
        oh[...] = x0
        oy[...] = dinv * x0

    sspec = pl.BlockSpec((2, BLK, 128), lambda i: (0, i, 0))
    cspec = pl.BlockSpec((2, BLK, 128), lambda i: (0, i, 0))
    ospec = pl.BlockSpec((BLK, 128), lambda i: (i, 0))
    return pl.pallas_call(
        body,
        grid=grid,
        in_specs=[
            sspec, cspec, sspec, cspec,
            pl.BlockSpec((BLK, 128), lambda i: (i, 0)),
            pl.BlockSpec((BLK, 128), lambda i: (i, 0)),
            pl.BlockSpec((1, 128), lambda i: (0, 0)),
        ],
        out_specs=[ospec] * 2,
        out_shape=[jax.ShapeDtypeStruct((N_TABLE, 128), jnp.float32)] * 2,
    )(s2rev, cnt_rev, s2tt, cnt_tt, tab1r, z0, bias)


def _tc_appnp_step(part, cnt_tt, y, h):
    grid = (6,)

    def body(p_ref, ct_ref, y_ref, h_ref, oz, oy):
        ct = ct_ref[0, :, 0:1] + ct_ref[1, :, 0:1]
        dinv = lax.rsqrt(ct + 1.0)
        psum = p_ref[0] + p_ref[1]
        z = (1.0 - ALPHA) * dinv * (psum + y_ref[...]) + ALPHA * h_ref[...]
        oz[...] = z
        oy[...] = dinv * z

    sspec = pl.BlockSpec((2, BLK, 128), lambda i: (0, i, 0))
    cspec = pl.BlockSpec((2, BLK, 128), lambda i: (0, i, 0))
    ospec = pl.BlockSpec((BLK, 128), lambda i: (i, 0))
    return pl.pallas_call(
        body,
        grid=grid,
        in_specs=[
            sspec, cspec,
            pl.BlockSpec((BLK, 128), lambda i: (i, 0)),
            pl.BlockSpec((BLK, 128), lambda i: (i, 0)),
        ],
        out_specs=[ospec] * 2,
        out_shape=[jax.ShapeDtypeStruct((N_TABLE, 128), jnp.float32)] * 2,
    )(part, cnt_tt, y, h)


def _tc_scores(z, wo_row, bo):
    grid = (6,)

    def body(z_ref, w_ref, b_ref, out):
        s = jnp.sum(z_ref[...] * w_ref[...], axis=1, keepdims=True) + b_ref[...]
        out[...] = jnp.broadcast_to(s, (BLK, 128))

    return pl.pallas_call(
        body,
        grid=grid,
        in_specs=[
            pl.BlockSpec((BLK, 128), lambda i: (i, 0)),
            pl.BlockSpec((1, 128), lambda i: (0, 0)),
            pl.BlockSpec((1, 1), lambda i: (0, 0)),
        ],
        out_specs=pl.BlockSpec((BLK, 128), lambda i: (i, 0)),
        out_shape=jax.ShapeDtypeStruct((N_TABLE, 128), jnp.float32),
    )(z, wo_row, bo)


def kernel(x_table, x_column, query_vec, hc_src, hc_dst, rev_src, rev_dst,
           cs_src, cs_dst, tt_src, tt_dst, params):
    p = params
    f32 = jnp.float32

    wcat1 = jnp.concatenate(
        [p["Wl1_hc"], p["Wl1_tt"], p["Wr1_rev"] + p["Wr1_tt"],
         query_vec[:, None], jnp.zeros((512, 127), f32)], axis=1)
    qv2 = query_vec[None, :]
    bq2 = p["bq"][None, :]
    o_hc, o_tt, xtr, z0 = _tc_table_proj(x_table, wcat1, qv2, p["Wq"], bq2)

    wcat2 = jnp.concatenate(
        [p["Wl1_cs"], p["Wl1_rev"], p["Wr1_hc"] + p["Wr1_cs"]], axis=1)
    o_cs, o_rev, xcr = _tc_col_proj(x_column, wcat2)

    hs, hd = _pad_edges(hc_src, hc_dst, N_COL, 4096)
    css, csd = _pad_edges(cs_src, cs_dst, N_COL, 4096)
    rs, rd = _pad_edges(rev_src, rev_dst, N_TABLE, 8192)
    ts, td = _pad_edges(tt_src, tt_dst, N_TABLE, 8192)

    onesd = jnp.ones((8, 128), f32)
    cnt_hc = _seg_win(onesd, hs, hd, ones_mode=True)
    cnt_cs = _seg_win(onesd, css, csd, ones_mode=True)
    cnt_rev = _seg_table(onesd, rs, rd, ones_mode=True)
    cnt_tt = _seg_table(onesd, ts, td, ones_mode=True)
    sum_hc = _seg_win(o_hc, hs, hd)
    sum_cs = _seg_win(o_cs, css, csd)
    sum_rev = _seg_table(o_rev, rs, rd)
    sum_tt = _seg_table(o_tt, ts, td)

    bias1c = (p["b1_hc"] + p["b1_cs"])[None, :]
    col1p = _tc_col1(sum_hc, cnt_hc, sum_cs, cnt_cs, xcr, bias1c, p["Wl2_rev"])

    bias1t = (p["b1_rev"] + p["b1_tt"])[None, :]
    wcat3 = jnp.concatenate([p["Wl2_tt"], p["Wr2_rev"] + p["Wr2_tt"]], axis=1)
    tab1p, tab1r = _tc_tab1(sum_rev, cnt_rev, sum_tt, cnt_tt, xtr, bias1t, wcat3)

    s2rev = _seg_table(col1p, rs, rd)
    s2tt = _seg_table(tab1p, ts, td)

    bias2t = (p["b2_rev"] + p["b2_tt"])[None, :]
    h, y = _tc_x0(s2rev, cnt_rev, s2tt, cnt_tt, tab1r, z0, bias2t)

    z = h
    for _ in range(K_APPNP):
        part = _seg_table(y, ts, td)
        z, y = _tc_appnp_step(part, cnt_tt, y, h)

    wo_row = p["Wo"].reshape(1, 128)
    bo = p["bo"].reshape(1, 1)
    sb = _tc_scores(z, wo_row, bo)
    return sb[:, 0]
